# baseline (speedup 1.0000x reference)
import jax
import jax.numpy as jnp
from jax import lax
from jax.experimental import pallas as pl
from jax.experimental.pallas import tpu as pltpu
from jax.experimental.pallas import tpu_sc as plsc

BATCH = 16384
D = 32
HID = 64
NC, NS = 2, 16
NW = NC * NS
ROWS_PER_W = BATCH // NW
CHUNK = 128
NCHUNK = ROWS_PER_W // CHUNK
IDX_ROWS = BATCH // CHUNK
LANES = 16
NGROUP = CHUNK // LANES


def _gather_body(uidx_hbm, midx_hbm, u_tab, m_tab, u_out, m_out,
                 uidx_v, midx_v, ubuf, mbuf, sem):
    wid = lax.axis_index("s") * NC + lax.axis_index("c")
    base = wid * NCHUNK
    pltpu.sync_copy(uidx_hbm.at[pl.ds(base, NCHUNK)], uidx_v)
    pltpu.sync_copy(midx_hbm.at[pl.ds(base, NCHUNK)], midx_v)
    g = []
    for j in range(NCHUNK):
        p = j % 2
        if j >= 2:
            g[j - 2][0].wait()
            g[j - 2][1].wait()
            pltpu.sync_copy(ubuf.at[p], u_out.at[base + j - 2])
            pltpu.sync_copy(mbuf.at[p], m_out.at[base + j - 2])
        g.append((pltpu.async_copy(u_tab.at[uidx_v.at[j]], ubuf.at[p], sem),
                  pltpu.async_copy(m_tab.at[midx_v.at[j]], mbuf.at[p], sem)))
    for j in (NCHUNK - 2, NCHUNK - 1):
        p = j % 2
        g[j][0].wait()
        g[j][1].wait()
        pltpu.sync_copy(ubuf.at[p], u_out.at[base + j])
        pltpu.sync_copy(mbuf.at[p], m_out.at[base + j])


def _sc_gather(uidx, midx, u_tab, m_tab):
    mesh = plsc.VectorSubcoreMesh(core_axis_name="c", subcore_axis_name="s",
                                  num_cores=NC, num_subcores=NS)
    out_t = (jax.ShapeDtypeStruct((IDX_ROWS, CHUNK, D), jnp.bfloat16),
             jax.ShapeDtypeStruct((IDX_ROWS, CHUNK, D), jnp.bfloat16))
    scratch = [
        pltpu.VMEM((NCHUNK, CHUNK), jnp.int32),
        pltpu.VMEM((NCHUNK, CHUNK), jnp.int32),
        pltpu.VMEM((2, CHUNK, D), jnp.bfloat16),
        pltpu.VMEM((2, CHUNK, D), jnp.bfloat16),
        pltpu.SemaphoreType.DMA,
    ]
    params = pltpu.CompilerParams(use_tc_tiling_on_sc=False)
    return pl.kernel(_gather_body, out_type=out_t, mesh=mesh,
                     scratch_types=scratch,
                     compiler_params=params)(uidx, midx, u_tab, m_tab)


def _mlp_body(u_ref, m_ref, w1_ref, b1_ref, w2_ref, b2_ref, o_ref):
    h = jnp.dot(u_ref[...], w1_ref[0:D, :], preferred_element_type=jnp.float32)
    h = h + jnp.dot(m_ref[...], w1_ref[D:2 * D, :],
                    preferred_element_type=jnp.float32)
    h = jnp.maximum(h + b1_ref[...], 0.0)
    o = jnp.dot(h, w2_ref[...], preferred_element_type=jnp.float32) + b2_ref[...]
    o_ref[...] = jax.nn.sigmoid(o) * 5.5


def _mlp(u_emb, m_emb, w1, b1, w2, b2, block_rows=2048):
    grid = (BATCH // block_rows,)
    return pl.pallas_call(
        _mlp_body,
        grid=grid,
        in_specs=[
            pl.BlockSpec((block_rows, D), lambda i: (i, 0)),
            pl.BlockSpec((block_rows, D), lambda i: (i, 0)),
            pl.BlockSpec((2 * D, HID), lambda i: (0, 0)),
            pl.BlockSpec((1, HID), lambda i: (0, 0)),
            pl.BlockSpec((HID, 1), lambda i: (0, 0)),
            pl.BlockSpec((1, 1), lambda i: (0, 0)),
        ],
        out_specs=pl.BlockSpec((block_rows, 1), lambda i: (i, 0)),
        out_shape=jax.ShapeDtypeStruct((BATCH, 1), jnp.float32),
    )(u_emb, m_emb, w1, b1.reshape(1, HID), w2, b2.reshape(1, 1))


def kernel(cats, u_table, m_table, w1, b1, w2, b2):
    cats = cats.astype(jnp.int32)
    uidx = cats[:, 0].reshape(IDX_ROWS, CHUNK)
    midx = cats[:, 1].reshape(IDX_ROWS, CHUNK)
    u_emb, m_emb = _sc_gather(uidx, midx,
                              u_table.astype(jnp.bfloat16),
                              m_table.astype(jnp.bfloat16))
    u_emb = u_emb.reshape(BATCH, D)
    m_emb = m_emb.reshape(BATCH, D)
    return _mlp(u_emb, m_emb, w1, b1, w2, b2)


# speedup vs baseline: 1.7439x; 1.7439x over previous
import jax
import jax.numpy as jnp
from jax import lax
from jax.experimental import pallas as pl
from jax.experimental.pallas import tpu as pltpu
from jax.experimental.pallas import tpu_sc as plsc

BATCH = 16384
D = 32
HID = 64
NC, NS = 2, 16
NW = NC * NS
ROWS_PER_W = BATCH // NW
CHUNK = 128
NCHUNK = ROWS_PER_W // CHUNK
IDX_ROWS = BATCH // CHUNK
LANES = 16
NGROUP = CHUNK // LANES


def _gather_body(idx_hbm, tab, out, idx_v, buf, sem):
    wid = lax.axis_index("s") * NC + lax.axis_index("c")
    base = wid * ROWS_PER_W
    pltpu.sync_copy(idx_hbm.at[pl.ds(base, ROWS_PER_W)], idx_v)

    def fire(j, p):
        def group(g, carry):
            off = pl.multiple_of(j * CHUNK + g * LANES, LANES)
            v = idx_v[pl.ds(off, LANES)]
            for l in range(LANES):
                r = v[l]
                row = g * LANES + l
                pltpu.make_async_copy(tab.at[pl.ds(r, 1)],
                                      buf.at[p, pl.ds(row, 1)], sem).start()
            return carry
        lax.fori_loop(0, NGROUP, group, 0)

    def drain_and_copy_out(j, p):
        out_row = wid * NCHUNK + j
        pltpu.make_async_copy(out.at[out_row], buf.at[p], sem).wait()
        pltpu.sync_copy(buf.at[p], out.at[out_row])

    fire(0, 0)
    for j in range(1, NCHUNK):
        fire(j, j % 2)
        drain_and_copy_out(j - 1, (j - 1) % 2)
    drain_and_copy_out(NCHUNK - 1, (NCHUNK - 1) % 2)


def _sc_gather(idx, tab):
    mesh = plsc.VectorSubcoreMesh(core_axis_name="c", subcore_axis_name="s",
                                  num_cores=NC, num_subcores=NS)
    out_t = jax.ShapeDtypeStruct((IDX_ROWS, CHUNK, D), jnp.float32)
    scratch = [
        pltpu.VMEM((ROWS_PER_W,), jnp.int32),
        pltpu.VMEM((2, CHUNK, D), jnp.float32),
        pltpu.SemaphoreType.DMA,
    ]
    return pl.kernel(_gather_body, out_type=out_t, mesh=mesh,
                     scratch_types=scratch)(idx, tab)


def _mlp_body(u_ref, m_ref, w1_ref, b1_ref, w2_ref, b2_ref, o_ref):
    h = jnp.dot(u_ref[...], w1_ref[0:D, :], preferred_element_type=jnp.float32)
    h = h + jnp.dot(m_ref[...], w1_ref[D:2 * D, :],
                    preferred_element_type=jnp.float32)
    h = jnp.maximum(h + b1_ref[...], 0.0)
    o = jnp.dot(h, w2_ref[...], preferred_element_type=jnp.float32) + b2_ref[...]
    o_ref[...] = jax.nn.sigmoid(o) * 5.5


def _mlp(u_emb, m_emb, w1, b1, w2, b2, block_rows=2048):
    grid = (BATCH // block_rows,)
    return pl.pallas_call(
        _mlp_body,
        grid=grid,
        in_specs=[
            pl.BlockSpec((block_rows, D), lambda i: (i, 0)),
            pl.BlockSpec((block_rows, D), lambda i: (i, 0)),
            pl.BlockSpec((2 * D, HID), lambda i: (0, 0)),
            pl.BlockSpec((1, HID), lambda i: (0, 0)),
            pl.BlockSpec((HID, 1), lambda i: (0, 0)),
            pl.BlockSpec((1, 1), lambda i: (0, 0)),
        ],
        out_specs=pl.BlockSpec((block_rows, 1), lambda i: (i, 0)),
        out_shape=jax.ShapeDtypeStruct((BATCH, 1), jnp.float32),
    )(u_emb, m_emb, w1, b1.reshape(1, HID), w2, b2.reshape(1, 1))


def kernel(cats, u_table, m_table, w1, b1, w2, b2):
    cats = cats.astype(jnp.int32)
    uidx = cats[:, 0]
    midx = cats[:, 1]
    u_emb = _sc_gather(uidx, u_table)
    m_emb = _sc_gather(midx, m_table)
    u_emb = u_emb.reshape(BATCH, D)
    m_emb = m_emb.reshape(BATCH, D)
    return _mlp(u_emb, m_emb, w1, b1, w2, b2)


# speedup vs baseline: 1.7440x; 1.0000x over previous
import jax
import jax.numpy as jnp
from jax import lax
from jax.experimental import pallas as pl
from jax.experimental.pallas import tpu as pltpu
from jax.experimental.pallas import tpu_sc as plsc

BATCH = 16384
D = 32
HID = 64
NC, NS = 2, 16
NW = NC * NS
ROWS_PER_W = BATCH // NW
CHUNK = 128
NCHUNK = ROWS_PER_W // CHUNK
IDX_ROWS = BATCH // CHUNK
LANES = 16
NGROUP = CHUNK // LANES


def _gather_body(idx_hbm, tab, out, idx_v, buf, sem0, sem1):
    wid = lax.axis_index("s") * NC + lax.axis_index("c")
    base = wid * ROWS_PER_W
    pltpu.sync_copy(idx_hbm.at[pl.ds(base, ROWS_PER_W)], idx_v)
    sems = (sem0, sem1)

    def fire(j, p):
        def group(g, carry):
            off = pl.multiple_of(j * CHUNK + g * LANES, LANES)
            v = idx_v[pl.ds(off, LANES)]
            for l in range(LANES):
                r = v[l]
                row = g * LANES + l
                pltpu.make_async_copy(tab.at[pl.ds(r, 1)],
                                      buf.at[p, pl.ds(row, 1)],
                                      sems[p]).start()
            return carry
        lax.fori_loop(0, NGROUP, group, 0)

    def drain_and_copy_out(j, p):
        out_row = wid * NCHUNK + j
        pltpu.make_async_copy(out.at[out_row], buf.at[p], sems[p]).wait()
        pltpu.sync_copy(buf.at[p], out.at[out_row])

    fire(0, 0)
    for j in range(1, NCHUNK):
        fire(j, j % 2)
        drain_and_copy_out(j - 1, (j - 1) % 2)
    drain_and_copy_out(NCHUNK - 1, (NCHUNK - 1) % 2)


def _sc_gather(idx, tab):
    mesh = plsc.VectorSubcoreMesh(core_axis_name="c", subcore_axis_name="s",
                                  num_cores=NC, num_subcores=NS)
    out_t = jax.ShapeDtypeStruct((IDX_ROWS, CHUNK, D), jnp.float32)
    scratch = [
        pltpu.VMEM((ROWS_PER_W,), jnp.int32),
        pltpu.VMEM((2, CHUNK, D), jnp.float32),
        pltpu.SemaphoreType.DMA,
        pltpu.SemaphoreType.DMA,
    ]
    return pl.kernel(_gather_body, out_type=out_t, mesh=mesh,
                     scratch_types=scratch)(idx, tab)


def _mlp_body(u_ref, m_ref, w1_ref, b1_ref, w2_ref, b2_ref, o_ref):
    h = jnp.dot(u_ref[...], w1_ref[0:D, :], preferred_element_type=jnp.float32)
    h = h + jnp.dot(m_ref[...], w1_ref[D:2 * D, :],
                    preferred_element_type=jnp.float32)
    h = jnp.maximum(h + b1_ref[...], 0.0)
    o = jnp.dot(h, w2_ref[...], preferred_element_type=jnp.float32) + b2_ref[...]
    o_ref[...] = jax.nn.sigmoid(o) * 5.5


def _mlp(u_emb, m_emb, w1, b1, w2, b2, block_rows=2048):
    grid = (BATCH // block_rows,)
    return pl.pallas_call(
        _mlp_body,
        grid=grid,
        in_specs=[
            pl.BlockSpec((block_rows, D), lambda i: (i, 0)),
            pl.BlockSpec((block_rows, D), lambda i: (i, 0)),
            pl.BlockSpec((2 * D, HID), lambda i: (0, 0)),
            pl.BlockSpec((1, HID), lambda i: (0, 0)),
            pl.BlockSpec((HID, 1), lambda i: (0, 0)),
            pl.BlockSpec((1, 1), lambda i: (0, 0)),
        ],
        out_specs=pl.BlockSpec((block_rows, 1), lambda i: (i, 0)),
        out_shape=jax.ShapeDtypeStruct((BATCH, 1), jnp.float32),
    )(u_emb, m_emb, w1, b1.reshape(1, HID), w2, b2.reshape(1, 1))


def kernel(cats, u_table, m_table, w1, b1, w2, b2):
    cats = cats.astype(jnp.int32)
    uidx = cats[:, 0]
    midx = cats[:, 1]
    u_emb = _sc_gather(uidx, u_table)
    m_emb = _sc_gather(midx, m_table)
    u_emb = u_emb.reshape(BATCH, D)
    m_emb = m_emb.reshape(BATCH, D)
    return _mlp(u_emb, m_emb, w1, b1, w2, b2)
